# baseline (speedup 1.0000x reference)
import functools

import jax
import jax.numpy as jnp
from jax import lax
from jax.experimental import pallas as pl
from jax.experimental.pallas import tpu as pltpu
from jax.experimental.pallas import tpu_sc as plsc

B = 16384
HIST = 200
D = 64
OUT = 50
VOCAB = 100000

NC = 2
NS = 16
NW = NC * NS
LANES = 16

S_BLK = 8
ROWS = S_BLK * HIST
PER_W = B // NW
NBLK = PER_W // S_BLK
C16 = D // LANES


def _sc_body(sent_hbm, table_hbm, out_hbm,
             idx0, idx1, rows0, rows1, outv0, outv1,
             gsem0, gsem1, isem0, isem1, osem0, osem1):
    cid = lax.axis_index("c")
    sid = lax.axis_index("s")
    wid = sid * NC + cid
    out_base = wid * PER_W

    idx_bufs = (idx0, idx1)
    rows_bufs = (rows0, rows1)
    out_bufs = (outv0, outv1)
    gsems = (gsem0, gsem1)
    isems = (isem0, isem1)
    osems = (osem0, osem1)

    def idx_copy(blk, b):
        pltpu.async_copy(
            sent_hbm.at[pl.ds(out_base + blk * S_BLK, S_BLK), :],
            idx_bufs[b], isems[b])

    def idx_wait(b):
        pltpu.make_async_copy(sent_hbm.at[pl.ds(out_base, S_BLK), :],
                              idx_bufs[b], isems[b]).wait()

    def gather(b):
        for s in range(S_BLK):
            pltpu.async_copy(table_hbm.at[idx_bufs[b].at[s]],
                             rows_bufs[b].at[s], gsems[b])

    def gather_wait(b, s):
        pltpu.make_async_copy(table_hbm.at[idx_bufs[b].at[s]],
                              rows_bufs[b].at[s], gsems[b]).wait()

    idx_copy(0, 0)
    idx_wait(0)
    gather(0)
    idx_copy(1, 1)

    def outer(g, carry):
        for b in range(2):
            blk = g * 2 + b

            @pl.when(blk + 1 < NBLK)
            def _():
                idx_wait(1 - b)
                gather(1 - b)

            @pl.when(blk >= 2)
            def _():
                pltpu.make_async_copy(
                    out_bufs[b],
                    out_hbm.at[pl.ds(out_base, S_BLK)], osems[b]).wait()

            rows = rows_bufs[b]
            outv = out_bufs[b]
            for s in range(S_BLK):
                gather_wait(b, s)

                def racc(p, accs, s=s, rows=rows):
                    row = 2 * p
                    new = list(accs)
                    for h in range(D // 32):
                        x0 = rows[s, row, pl.ds(h * 32, 32)]
                        x1 = rows[s, row + 1, pl.ds(h * 32, 32)]
                        lo, hi = plsc.unpack(
                            x0 + x1, format=plsc.PackFormat.INTERLEAVED)
                        new[2 * h] = new[2 * h] + lo
                        new[2 * h + 1] = new[2 * h + 1] + hi
                    return tuple(new)

                accs = lax.fori_loop(
                    0, HIST // 2, racc,
                    tuple(jnp.zeros((LANES,), jnp.float32) for _ in range(C16)),
                    unroll=10)
                for c in range(C16):
                    outv[s, pl.ds(c * LANES, LANES)] = accs[c]

                if s == S_BLK - 1:
                    @pl.when(blk + 2 < NBLK)
                    def _():
                        idx_copy(blk + 2, b)

            pltpu.async_copy(outv,
                             out_hbm.at[pl.ds(out_base + blk * S_BLK, S_BLK)],
                             osems[b])
        return carry

    lax.fori_loop(0, NBLK // 2, outer, 0)

    for b in range(2):
        pltpu.make_async_copy(out_bufs[b],
                              out_hbm.at[pl.ds(out_base, S_BLK)],
                              osems[b]).wait()


_sc_gather_sum = functools.partial(
    pl.kernel,
    out_type=jax.ShapeDtypeStruct((B, D), jnp.float32),
    mesh=plsc.VectorSubcoreMesh(core_axis_name="c", subcore_axis_name="s"),
    compiler_params=pltpu.CompilerParams(use_tc_tiling_on_sc=False,
                                         needs_layout_passes=False),
    scratch_types=[
        pltpu.VMEM((S_BLK, HIST), jnp.int32),
        pltpu.VMEM((S_BLK, HIST), jnp.int32),
        pltpu.VMEM((S_BLK, HIST, D), jnp.bfloat16),
        pltpu.VMEM((S_BLK, HIST, D), jnp.bfloat16),
        pltpu.VMEM((S_BLK, D), jnp.float32),
        pltpu.VMEM((S_BLK, D), jnp.float32),
        pltpu.SemaphoreType.DMA,
        pltpu.SemaphoreType.DMA,
        pltpu.SemaphoreType.DMA,
        pltpu.SemaphoreType.DMA,
        pltpu.SemaphoreType.DMA,
        pltpu.SemaphoreType.DMA,
    ],
)(_sc_body)


CAST_BLK = 2000


def _cast_body(x_ref, o_ref):
    o_ref[...] = x_ref[...].astype(jnp.bfloat16).reshape(-1)


def _cast_table(table):
    return pl.pallas_call(
        _cast_body,
        grid=(VOCAB // CAST_BLK,),
        in_specs=[pl.BlockSpec((CAST_BLK, D), lambda i: (i, 0))],
        out_specs=pl.BlockSpec((CAST_BLK * D,), lambda i: (i,)),
        out_shape=jax.ShapeDtypeStruct((VOCAB * D,), jnp.bfloat16),
    )(table)


def _mm_body(x_ref, w_ref, b_ref, o_ref):
    o_ref[...] = (jnp.dot(x_ref[...], w_ref[...],
                          preferred_element_type=jnp.float32)
                  * (1.0 / HIST) + b_ref[...])


MM_BLK = 2048


def _mean_matmul(sums, W, b2):
    return pl.pallas_call(
        _mm_body,
        grid=(B // MM_BLK,),
        in_specs=[
            pl.BlockSpec((MM_BLK, D), lambda i: (i, 0)),
            pl.BlockSpec((D, OUT), lambda i: (0, 0)),
            pl.BlockSpec((1, OUT), lambda i: (0, 0)),
        ],
        out_specs=pl.BlockSpec((MM_BLK, OUT), lambda i: (i, 0)),
        out_shape=jax.ShapeDtypeStruct((B, OUT), jnp.float32),
    )(sums, W, b2)


_PERM = [32 * (k // 2) + 2 * l + (k % 2) for k in range(4) for l in range(16)]


def kernel(sentence, table, W, b):
    sums = _sc_gather_sum(sentence.astype(jnp.int32),
                          table.astype(jnp.bfloat16))
    W2 = W[jnp.array(_PERM, dtype=jnp.int32), :]
    return _mean_matmul(sums, W2, b.reshape(1, OUT))


# speedup vs baseline: 1.0182x; 1.0182x over previous
import functools

import jax
import jax.numpy as jnp
from jax import lax
from jax.experimental import pallas as pl
from jax.experimental.pallas import tpu as pltpu
from jax.experimental.pallas import tpu_sc as plsc

B = 16384
HIST = 200
D = 64
OUT = 50
VOCAB = 100000

NC = 2
NS = 16
NW = NC * NS
LANES = 16

S_BLK = 8
ROWS = S_BLK * HIST
PER_W = B // NW
NBLK = PER_W // S_BLK
C16 = D // LANES


def _sc_body(sent_hbm, table_hbm, out_hbm,
             idx0, idx1, rows0, rows1, outv0, outv1,
             gsem0, gsem1, isem0, isem1, osem0, osem1):
    cid = lax.axis_index("c")
    sid = lax.axis_index("s")
    wid = sid * NC + cid
    out_base = wid * PER_W

    idx_bufs = (idx0, idx1)
    rows_bufs = (rows0, rows1)
    out_bufs = (outv0, outv1)
    gsems = (gsem0, gsem1)
    isems = (isem0, isem1)
    osems = (osem0, osem1)

    def idx_copy(blk, b):
        pltpu.async_copy(
            sent_hbm.at[pl.ds(out_base + blk * S_BLK, S_BLK), :],
            idx_bufs[b], isems[b])

    def idx_wait(b):
        pltpu.make_async_copy(sent_hbm.at[pl.ds(out_base, S_BLK), :],
                              idx_bufs[b], isems[b]).wait()

    def gather(b):
        for s in range(S_BLK):
            pltpu.async_copy(table_hbm.at[idx_bufs[b].at[s]],
                             rows_bufs[b].at[s], gsems[b])

    def gather_wait(b, s):
        pltpu.make_async_copy(table_hbm.at[idx_bufs[b].at[s]],
                              rows_bufs[b].at[s], gsems[b]).wait()

    idx_copy(0, 0)
    idx_wait(0)
    gather(0)
    idx_copy(1, 1)

    def outer(g, carry):
        for b in range(2):
            blk = g * 2 + b

            @pl.when(blk + 1 < NBLK)
            def _():
                idx_wait(1 - b)
                gather(1 - b)

            @pl.when(blk >= 2)
            def _():
                pltpu.make_async_copy(
                    out_bufs[b],
                    out_hbm.at[pl.ds(out_base, S_BLK)], osems[b]).wait()

            rows = rows_bufs[b]
            outv = out_bufs[b]
            for s in range(S_BLK):
                gather_wait(b, s)

                def racc(p, accs, s=s, rows=rows):
                    row = 2 * p
                    new = list(accs)
                    for h in range(D // 32):
                        x0 = rows[s, row, pl.ds(h * 32, 32)]
                        x1 = rows[s, row + 1, pl.ds(h * 32, 32)]
                        lo, hi = plsc.unpack(
                            x0 + x1, format=plsc.PackFormat.INTERLEAVED)
                        new[2 * h] = new[2 * h] + lo
                        new[2 * h + 1] = new[2 * h + 1] + hi
                    return tuple(new)

                accs = lax.fori_loop(
                    0, HIST // 2, racc,
                    tuple(jnp.zeros((LANES,), jnp.float32) for _ in range(C16)),
                    unroll=4)
                for c in range(C16):
                    outv[s, pl.ds(c * LANES, LANES)] = accs[c]

                if s == S_BLK - 1:
                    @pl.when(blk + 2 < NBLK)
                    def _():
                        idx_copy(blk + 2, b)

            pltpu.async_copy(outv,
                             out_hbm.at[pl.ds(out_base + blk * S_BLK, S_BLK)],
                             osems[b])
        return carry

    lax.fori_loop(0, NBLK // 2, outer, 0)

    for b in range(2):
        pltpu.make_async_copy(out_bufs[b],
                              out_hbm.at[pl.ds(out_base, S_BLK)],
                              osems[b]).wait()


_sc_gather_sum = functools.partial(
    pl.kernel,
    out_type=jax.ShapeDtypeStruct((B, D), jnp.float32),
    mesh=plsc.VectorSubcoreMesh(core_axis_name="c", subcore_axis_name="s"),
    compiler_params=pltpu.CompilerParams(use_tc_tiling_on_sc=False,
                                         needs_layout_passes=False),
    scratch_types=[
        pltpu.VMEM((S_BLK, HIST), jnp.int32),
        pltpu.VMEM((S_BLK, HIST), jnp.int32),
        pltpu.VMEM((S_BLK, HIST, D), jnp.bfloat16),
        pltpu.VMEM((S_BLK, HIST, D), jnp.bfloat16),
        pltpu.VMEM((S_BLK, D), jnp.float32),
        pltpu.VMEM((S_BLK, D), jnp.float32),
        pltpu.SemaphoreType.DMA,
        pltpu.SemaphoreType.DMA,
        pltpu.SemaphoreType.DMA,
        pltpu.SemaphoreType.DMA,
        pltpu.SemaphoreType.DMA,
        pltpu.SemaphoreType.DMA,
    ],
)(_sc_body)


CAST_BLK = 2000


def _cast_body(x_ref, o_ref):
    o_ref[...] = x_ref[...].astype(jnp.bfloat16).reshape(-1)


def _cast_table(table):
    return pl.pallas_call(
        _cast_body,
        grid=(VOCAB // CAST_BLK,),
        in_specs=[pl.BlockSpec((CAST_BLK, D), lambda i: (i, 0))],
        out_specs=pl.BlockSpec((CAST_BLK * D,), lambda i: (i,)),
        out_shape=jax.ShapeDtypeStruct((VOCAB * D,), jnp.bfloat16),
    )(table)


def _mm_body(x_ref, w_ref, b_ref, o_ref):
    o_ref[...] = (jnp.dot(x_ref[...], w_ref[...],
                          preferred_element_type=jnp.float32)
                  * (1.0 / HIST) + b_ref[...])


MM_BLK = 4096


def _mean_matmul(sums, W, b2):
    return pl.pallas_call(
        _mm_body,
        grid=(B // MM_BLK,),
        in_specs=[
            pl.BlockSpec((MM_BLK, D), lambda i: (i, 0)),
            pl.BlockSpec((D, OUT), lambda i: (0, 0)),
            pl.BlockSpec((1, OUT), lambda i: (0, 0)),
        ],
        out_specs=pl.BlockSpec((MM_BLK, OUT), lambda i: (i, 0)),
        out_shape=jax.ShapeDtypeStruct((B, OUT), jnp.float32),
    )(sums, W, b2)


_PERM = [32 * (k // 2) + 2 * l + (k % 2) for k in range(4) for l in range(16)]


def kernel(sentence, table, W, b):
    sums = _sc_gather_sum(sentence.astype(jnp.int32),
                          table.astype(jnp.bfloat16))
    W2 = W[jnp.array(_PERM, dtype=jnp.int32), :]
    return _mean_matmul(sums, W2, b.reshape(1, OUT))
